# baseline (device time: 16952 ns/iter reference)
import jax
import jax.numpy as jnp
from jax import lax
from jax.experimental import pallas as pl
from jax.experimental.pallas import tpu as pltpu

N_DEV = 8
N_HALF = 4
_XOR_FAR_FIRST = (6, 2, 5, 7, 1, 3, 4)


def kernel(A, B):
    m, _ = A.shape
    _, n = B.shape
    mc = m // N_DEV
    nh = n // N_HALF

    def body(a_ref, b_ref, out_ref, part_ref, rs_ref, ag_ref,
             rs_send_sems, rs_recv_sems, ag_send_sems, ag_recv_sems):
        my = lax.axis_index("i")

        def xor_swap(t, x):
            return jnp.concatenate(
                [t[j ^ x : (j ^ x) + 1] for j in range(N_DEV)], axis=0
            )

        def xor_permute(t):
            for bit in (4, 2, 1):
                t = jnp.where((my & bit) != 0, xor_swap(t, bit), t)
            return t

        barrier_sem = pltpu.get_barrier_semaphore()
        for p in range(N_DEV):

            @pl.when(p != my)
            def _(p=p):
                pl.semaphore_signal(
                    barrier_sem, inc=1,
                    device_id=(p,), device_id_type=pl.DeviceIdType.MESH,
                )

        a = xor_permute(
            a_ref[:, :].astype(jnp.bfloat16).reshape(N_DEV, mc, -1)
        ).reshape(m, -1)
        b = b_ref[:, :].astype(jnp.bfloat16)
        partial = jnp.dot(a, b, preferred_element_type=jnp.float32)
        for h in range(N_HALF):
            part_ref[h, :, :, :] = partial[:, h * nh:(h + 1) * nh].reshape(
                N_DEV, mc, nh
            ).astype(jnp.bfloat16)

        pl.semaphore_wait(barrier_sem, N_DEV - 1)

        rs_rdmas = {}
        for h in range(N_HALF):
            for x in _XOR_FAR_FIRST:
                rdma = pltpu.make_async_remote_copy(
                    src_ref=part_ref.at[h, x],
                    dst_ref=rs_ref.at[h, x],
                    send_sem=rs_send_sems.at[h, x],
                    recv_sem=rs_recv_sems.at[h, x],
                    device_id=(jnp.bitwise_xor(my, x),),
                    device_id_type=pl.DeviceIdType.MESH,
                )
                rs_rdmas[h, x] = rdma
                rdma.start()

        ag_rdmas = {}
        for h in range(N_HALF):
            rs_ref[h, 0, :, :] = part_ref[h, 0, :, :]

            zb = rs_ref[h, 0, :, :]
            for x in _XOR_FAR_FIRST[::-1]:
                rs_rdmas[h, x].wait_recv()
                zb = zb + rs_ref[h, x, :, :]

            z = zb.astype(jnp.float32)
            g = 0.5 * z * (
                1.0 + jnp.tanh(0.7978845608 * (z + 0.044715 * z * z * z))
            )
            ag_ref[h, 0, :, :] = g.astype(jnp.bfloat16)

            for x in _XOR_FAR_FIRST:
                rdma = pltpu.make_async_remote_copy(
                    src_ref=ag_ref.at[h, 0],
                    dst_ref=ag_ref.at[h, x],
                    send_sem=ag_send_sems.at[h, x],
                    recv_sem=ag_recv_sems.at[h, x],
                    device_id=(jnp.bitwise_xor(my, x),),
                    device_id_type=pl.DeviceIdType.MESH,
                )
                ag_rdmas[h, x] = rdma
                rdma.start()

        for h in range(N_HALF):
            for x in _XOR_FAR_FIRST[::-1]:
                ag_rdmas[h, x].wait_recv()
            out_ref[:, h * nh:(h + 1) * nh] = xor_permute(
                ag_ref[h, :, :, :]
            ).reshape(m, nh)

        for h in range(N_HALF):
            for x in _XOR_FAR_FIRST:
                rs_rdmas[h, x].wait_send()
                ag_rdmas[h, x].wait_send()

    return pl.pallas_call(
        body,
        out_shape=jax.ShapeDtypeStruct((m, n), jnp.bfloat16),
        in_specs=[
            pl.BlockSpec(memory_space=pltpu.VMEM),
            pl.BlockSpec(memory_space=pltpu.VMEM),
        ],
        out_specs=pl.BlockSpec(memory_space=pltpu.VMEM),
        scratch_shapes=[
            pltpu.VMEM((N_HALF, N_DEV, mc, nh), jnp.bfloat16),
            pltpu.VMEM((N_HALF, N_DEV, mc, nh), jnp.bfloat16),
            pltpu.VMEM((N_HALF, N_DEV, mc, nh), jnp.bfloat16),
            pltpu.SemaphoreType.DMA((N_HALF, N_DEV)),
            pltpu.SemaphoreType.DMA((N_HALF, N_DEV)),
            pltpu.SemaphoreType.DMA((N_HALF, N_DEV)),
            pltpu.SemaphoreType.DMA((N_HALF, N_DEV)),
        ],
        compiler_params=pltpu.CompilerParams(collective_id=0),
    )(A, B)


# device time: 16589 ns/iter; 1.0219x vs baseline; 1.0219x over previous
import jax
import jax.numpy as jnp
from jax import lax
from jax.experimental import pallas as pl
from jax.experimental.pallas import tpu as pltpu

N_DEV = 8
N_HALF = 2
_XOR_FAR_FIRST = (6, 2, 5, 7, 1, 3, 4)


def kernel(A, B):
    m, _ = A.shape
    _, n = B.shape
    mc = m // N_DEV
    nh = n // N_HALF

    def body(a_ref, b_ref, out_ref, part_ref, rs_ref, ag_ref,
             rs_send_sems, rs_recv_sems, ag_send_sems, ag_recv_sems):
        my = lax.axis_index("i")

        def xor_swap(t, x):
            return jnp.concatenate(
                [t[j ^ x : (j ^ x) + 1] for j in range(N_DEV)], axis=0
            )

        def xor_permute(t):
            for bit in (4, 2, 1):
                t = jnp.where((my & bit) != 0, xor_swap(t, bit), t)
            return t

        barrier_sem = pltpu.get_barrier_semaphore()
        for p in range(N_DEV):

            @pl.when(p != my)
            def _(p=p):
                pl.semaphore_signal(
                    barrier_sem, inc=1,
                    device_id=(p,), device_id_type=pl.DeviceIdType.MESH,
                )

        a = xor_permute(
            a_ref[:, :].astype(jnp.bfloat16).reshape(N_DEV, mc, -1)
        ).reshape(m, -1)
        b = b_ref[:, :].astype(jnp.bfloat16)
        partial = jnp.dot(a, b, preferred_element_type=jnp.float32)
        for h in range(N_HALF):
            part_ref[h, :, :, :] = partial[:, h * nh:(h + 1) * nh].reshape(
                N_DEV, mc, nh
            ).astype(jnp.bfloat16)

        pl.semaphore_wait(barrier_sem, N_DEV - 1)

        rs_rdmas = {}
        for h in range(N_HALF):
            for x in _XOR_FAR_FIRST:
                rdma = pltpu.make_async_remote_copy(
                    src_ref=part_ref.at[h, x],
                    dst_ref=rs_ref.at[h, x],
                    send_sem=rs_send_sems.at[h, x],
                    recv_sem=rs_recv_sems.at[h, x],
                    device_id=(jnp.bitwise_xor(my, x),),
                    device_id_type=pl.DeviceIdType.MESH,
                )
                rs_rdmas[h, x] = rdma
                rdma.start()

        ag_rdmas = {}
        for h in range(N_HALF):
            rs_ref[h, 0, :, :] = part_ref[h, 0, :, :]

            zb = rs_ref[h, 0, :, :]
            for x in _XOR_FAR_FIRST[::-1]:
                rs_rdmas[h, x].wait_recv()
                zb = zb + rs_ref[h, x, :, :]

            z = zb.astype(jnp.float32)
            g = 0.5 * z * (
                1.0 + jnp.tanh(0.7978845608 * (z + 0.044715 * z * z * z))
            )
            ag_ref[h, 0, :, :] = g.astype(jnp.bfloat16)

            for x in _XOR_FAR_FIRST:
                rdma = pltpu.make_async_remote_copy(
                    src_ref=ag_ref.at[h, 0],
                    dst_ref=ag_ref.at[h, x],
                    send_sem=ag_send_sems.at[h, x],
                    recv_sem=ag_recv_sems.at[h, x],
                    device_id=(jnp.bitwise_xor(my, x),),
                    device_id_type=pl.DeviceIdType.MESH,
                )
                ag_rdmas[h, x] = rdma
                rdma.start()

        for h in range(N_HALF):
            for x in _XOR_FAR_FIRST[::-1]:
                ag_rdmas[h, x].wait_recv()
            out_ref[:, h * nh:(h + 1) * nh] = xor_permute(
                ag_ref[h, :, :, :]
            ).reshape(m, nh)

        for h in range(N_HALF):
            for x in _XOR_FAR_FIRST:
                rs_rdmas[h, x].wait_send()
                ag_rdmas[h, x].wait_send()

    return pl.pallas_call(
        body,
        out_shape=jax.ShapeDtypeStruct((m, n), jnp.bfloat16),
        in_specs=[
            pl.BlockSpec(memory_space=pltpu.VMEM),
            pl.BlockSpec(memory_space=pltpu.VMEM),
        ],
        out_specs=pl.BlockSpec(memory_space=pltpu.VMEM),
        scratch_shapes=[
            pltpu.VMEM((N_HALF, N_DEV, mc, nh), jnp.bfloat16),
            pltpu.VMEM((N_HALF, N_DEV, mc, nh), jnp.bfloat16),
            pltpu.VMEM((N_HALF, N_DEV, mc, nh), jnp.bfloat16),
            pltpu.SemaphoreType.DMA((N_HALF, N_DEV)),
            pltpu.SemaphoreType.DMA((N_HALF, N_DEV)),
            pltpu.SemaphoreType.DMA((N_HALF, N_DEV)),
            pltpu.SemaphoreType.DMA((N_HALF, N_DEV)),
        ],
        compiler_params=pltpu.CompilerParams(collective_id=0),
    )(A, B)
